# baseline (speedup 1.0000x reference)
import functools

import jax
import jax.numpy as jnp
from jax import lax
from jax.experimental import pallas as pl
from jax.experimental.pallas import tpu as pltpu
from jax.experimental.pallas import tpu_sc as plsc

M, D = 131072, 128
TO_ADD_CONST = 1.0
SC_ROWS = 65536
TC_ROWS = M - SC_ROWS

NC, NS = 2, 16
NW = NC * NS
ROWS_PER_W = SC_ROWS // NW
SCHUNK = 256
NCH = ROWS_PER_W // SCHUNK
SNBUF = 4
CELEMS = SCHUNK * D

_mesh = plsc.VectorSubcoreMesh(core_axis_name="c", subcore_axis_name="s")


@functools.partial(
    pl.kernel,
    out_type=jax.ShapeDtypeStruct((SC_ROWS * D,), jnp.float32),
    mesh=_mesh,
    scratch_types=(
        [pltpu.VMEM((CELEMS,), jnp.float32)] * SNBUF
        + [pltpu.SemaphoreType.DMA] * (2 * SNBUF)
    ),
)
def _sc_copy(x_hbm, out_hbm, *rest):
    bufs = rest[:SNBUF]
    in_sems = rest[SNBUF:2 * SNBUF]
    out_sems = rest[2 * SNBUF:]

    wid = lax.axis_index("s") * NC + lax.axis_index("c")
    wbase = wid * (ROWS_PER_W * D)

    def in_cp(k):
        b = k % SNBUF
        return pltpu.make_async_copy(
            x_hbm.at[pl.ds(wbase + k * CELEMS, CELEMS)], bufs[b], in_sems[b])

    def out_cp(k):
        b = k % SNBUF
        return pltpu.make_async_copy(
            bufs[b], out_hbm.at[pl.ds(wbase + k * CELEMS, CELEMS)], out_sems[b])

    PREF = SNBUF // 2
    for j in range(PREF):
        in_cp(j).start()

    for k in range(NCH):
        in_cp(k).wait()
        out_cp(k).start()
        j = k + PREF
        if j < NCH:
            if j >= SNBUF:
                out_cp(j - SNBUF).wait()
            in_cp(j).start()

    for k in range(max(NCH - SNBUF, 0), NCH):
        out_cp(k).wait()


CHUNK = 8192
NCHUNKS = TC_ROWS // CHUNK
NBUF = 6


def _tc_body(x_hbm, o_hbm, *rest):
    bufs = rest[:NBUF]
    in_sems = rest[NBUF:2 * NBUF]
    out_sems = rest[2 * NBUF:3 * NBUF]

    def in_cp(k):
        b = k % NBUF
        return pltpu.make_async_copy(
            x_hbm.at[pl.ds(k * CHUNK, CHUNK), :], bufs[b], in_sems[b])

    def out_cp(k):
        b = k % NBUF
        return pltpu.make_async_copy(
            bufs[b], o_hbm.at[pl.ds(k * CHUNK, CHUNK), :], out_sems[b])

    for j in range(NBUF):
        in_cp(j).start()

    for k in range(NCHUNKS):
        in_cp(k).wait()
        out_cp(k).start()
        j = k + NBUF
        if j < NCHUNKS:
            out_cp(k).wait()
            in_cp(j).start()

    for k in range(max(NCHUNKS - NBUF, 0), NCHUNKS):
        out_cp(k).wait()


def _tc_copy(x_tc):
    return pl.pallas_call(
        _tc_body,
        grid=(1,),
        in_specs=[pl.BlockSpec(memory_space=pl.ANY)],
        out_specs=pl.BlockSpec(memory_space=pl.ANY),
        scratch_shapes=(
            [pltpu.VMEM((CHUNK, D), jnp.float32)] * NBUF
            + [pltpu.SemaphoreType.DMA] * (2 * NBUF)
        ),
        out_shape=jax.ShapeDtypeStruct((TC_ROWS, D), jnp.float32),
    )(x_tc)


def _fix_body(idx_ref, x_ref, o_ref):
    r = idx_ref[0] % 8
    o_ref[...] = x_ref[...]
    o_ref[pl.ds(r, 1), :] = x_ref[pl.ds(r, 1), :] + TO_ADD_CONST


def _fix_row(copied, idx_arr):
    grid_spec = pltpu.PrefetchScalarGridSpec(
        num_scalar_prefetch=1,
        grid=(1,),
        in_specs=[pl.BlockSpec((8, D), lambda i, idx: (idx[0] // 8, 0))],
        out_specs=pl.BlockSpec((8, D), lambda i, idx: (idx[0] // 8, 0)),
    )
    return pl.pallas_call(
        _fix_body,
        grid_spec=grid_spec,
        out_shape=jax.ShapeDtypeStruct((M, D), jnp.float32),
        input_output_aliases={1: 0},
    )(idx_arr, copied)


def kernel(tensor, slice_index, related_index):
    idx_arr = jnp.asarray(slice_index, dtype=jnp.int32).reshape((1,))
    sc_part = _sc_copy(tensor[:SC_ROWS].reshape(SC_ROWS * D)).reshape(SC_ROWS, D)
    tc_part = _tc_copy(tensor[SC_ROWS:])
    copied = jnp.concatenate([sc_part, tc_part], axis=0)
    out = _fix_row(copied, idx_arr)
    return (out, slice_index, related_index)


# speedup vs baseline: 2.3234x; 2.3234x over previous
import functools

import jax
import jax.numpy as jnp
from jax import lax
from jax.experimental import pallas as pl
from jax.experimental.pallas import tpu as pltpu
from jax.experimental.pallas import tpu_sc as plsc

M, D = 131072, 128
TO_ADD_CONST = 1.0
CHUNK = 8192
NCHUNKS = M // CHUNK
NBUF = 6
B = 8
L = 16


def _tc_body(x_hbm, o_hbm, *rest):
    bufs = rest[:NBUF]
    in_sems = rest[NBUF:2 * NBUF]
    out_sems = rest[2 * NBUF:3 * NBUF]

    def in_cp(k):
        b = k % NBUF
        return pltpu.make_async_copy(
            x_hbm.at[pl.ds(k * CHUNK, CHUNK), :], bufs[b], in_sems[b])

    def out_cp(k):
        b = k % NBUF
        return pltpu.make_async_copy(
            bufs[b], o_hbm.at[pl.ds(k * CHUNK, CHUNK), :], out_sems[b])

    for j in range(NBUF):
        in_cp(j).start()

    for k in range(NCHUNKS):
        in_cp(k).wait()
        out_cp(k).start()
        j = k + NBUF
        if j < NCHUNKS:
            out_cp(k).wait()
            in_cp(j).start()

    for k in range(max(NCHUNKS - NBUF, 0), NCHUNKS):
        out_cp(k).wait()


def _tc_copy(x):
    return pl.pallas_call(
        _tc_body,
        grid=(1,),
        in_specs=[pl.BlockSpec(memory_space=pl.ANY)],
        out_specs=pl.BlockSpec(memory_space=pl.ANY),
        scratch_shapes=(
            [pltpu.VMEM((CHUNK, D), jnp.float32)] * NBUF
            + [pltpu.SemaphoreType.DMA] * (2 * NBUF)
        ),
        out_shape=jax.ShapeDtypeStruct((M, D), jnp.float32),
    )(x)


_mesh = plsc.VectorSubcoreMesh(core_axis_name="c", subcore_axis_name="s")


@functools.partial(
    pl.kernel,
    out_type=(),
    mesh=_mesh,
    scratch_types=(
        pltpu.VMEM((B,), jnp.int32),
        pltpu.VMEM((B, D), jnp.float32),
        pltpu.SemaphoreType.DMA,
    ),
)
def _sc_fix(out_ref, idxv_hbm, idx_v, rows_v, sem):
    c = lax.axis_index("c")
    s = lax.axis_index("s")

    @pl.when((c == 0) & (s == 0))
    def _():
        pltpu.sync_copy(idxv_hbm, idx_v)
        pltpu.async_copy(out_ref.at[idx_v], rows_v, sem).wait()
        for i in range(B):
            for j in range(D // L):
                sl = (i, pl.ds(j * L, L))
                rows_v[sl] = rows_v[sl] + TO_ADD_CONST
        pltpu.async_copy(rows_v, out_ref.at[idx_v], sem).wait()


def kernel(tensor, slice_index, related_index):
    idxv = jnp.full((B,), slice_index, dtype=jnp.int32)
    copied = _tc_copy(tensor)
    ref = jax.new_ref(copied)
    _sc_fix(ref, idxv)
    out = ref[...]
    return (out, slice_index, related_index)


# speedup vs baseline: 2.3982x; 1.0322x over previous
import functools

import jax
import jax.numpy as jnp
from jax import lax
from jax.experimental import pallas as pl
from jax.experimental.pallas import tpu as pltpu
from jax.experimental.pallas import tpu_sc as plsc

M, D = 131072, 128
TO_ADD_CONST = 1.0
CHUNK = 8192
NCHUNKS = M // CHUNK
NBUF = 6
B = 8
L = 16


def _tc_body(x_hbm, o_hbm, *rest):
    bufs = rest[:NBUF]
    in_sems = rest[NBUF:2 * NBUF]
    out_sems = rest[2 * NBUF:3 * NBUF]

    def in_cp(k):
        b = k % NBUF
        return pltpu.make_async_copy(
            x_hbm.at[pl.ds(k * CHUNK, CHUNK), :], bufs[b], in_sems[b])

    def out_cp(k):
        b = k % NBUF
        return pltpu.make_async_copy(
            bufs[b], o_hbm.at[pl.ds(k * CHUNK, CHUNK), :], out_sems[b])

    for j in range(NBUF):
        in_cp(j).start()

    for k in range(NCHUNKS):
        in_cp(k).wait()
        out_cp(k).start()
        j = k + NBUF
        if j < NCHUNKS:
            out_cp(k).wait()
            in_cp(j).start()

    for k in range(max(NCHUNKS - NBUF, 0), NCHUNKS):
        out_cp(k).wait()


def _tc_copy(x):
    return pl.pallas_call(
        _tc_body,
        grid=(1,),
        in_specs=[pl.BlockSpec(memory_space=pl.ANY)],
        out_specs=pl.BlockSpec(memory_space=pl.ANY),
        scratch_shapes=(
            [pltpu.VMEM((CHUNK, D), jnp.float32)] * NBUF
            + [pltpu.SemaphoreType.DMA] * (2 * NBUF)
        ),
        out_shape=jax.ShapeDtypeStruct((M, D), jnp.float32),
    )(x)


_mesh = plsc.VectorSubcoreMesh(core_axis_name="c", subcore_axis_name="s")


@functools.partial(
    pl.kernel,
    out_type=jax.ShapeDtypeStruct((B, D), jnp.float32),
    mesh=_mesh,
    scratch_types=(
        pltpu.VMEM((B,), jnp.int32),
        pltpu.VMEM((B, D), jnp.float32),
        pltpu.SemaphoreType.DMA,
    ),
)
def _sc_gather_add(x_hbm, idxv_hbm, rows_out_hbm, idx_v, rows_v, sem):
    c = lax.axis_index("c")
    s = lax.axis_index("s")

    @pl.when((c == 0) & (s == 0))
    def _():
        pltpu.sync_copy(idxv_hbm, idx_v)
        pltpu.async_copy(x_hbm.at[idx_v], rows_v, sem).wait()
        for i in range(B):
            for j in range(D // L):
                sl = (i, pl.ds(j * L, L))
                rows_v[sl] = rows_v[sl] + TO_ADD_CONST
        pltpu.sync_copy(rows_v, rows_out_hbm)


def _fix_body(idx_ref, rows_ref, x_ref, o_ref):
    r = idx_ref[0] % 8
    o_ref[...] = x_ref[...]
    o_ref[pl.ds(r, 1), :] = rows_ref[pl.ds(0, 1), :]


def _fix_row(copied, rows, idx_arr):
    grid_spec = pltpu.PrefetchScalarGridSpec(
        num_scalar_prefetch=1,
        grid=(1,),
        in_specs=[
            pl.BlockSpec((B, D), lambda i, idx: (0, 0)),
            pl.BlockSpec((8, D), lambda i, idx: (idx[0] // 8, 0)),
        ],
        out_specs=pl.BlockSpec((8, D), lambda i, idx: (idx[0] // 8, 0)),
    )
    return pl.pallas_call(
        _fix_body,
        grid_spec=grid_spec,
        out_shape=jax.ShapeDtypeStruct((M, D), jnp.float32),
        input_output_aliases={2: 0},
    )(idx_arr, rows, copied)


def kernel(tensor, slice_index, related_index):
    idx_arr = jnp.asarray(slice_index, dtype=jnp.int32).reshape((1,))
    idxv = jnp.full((B,), slice_index, dtype=jnp.int32)
    rows = _sc_gather_add(tensor, idxv)
    copied = _tc_copy(tensor)
    out = _fix_row(copied, rows, idx_arr)
    return (out, slice_index, related_index)
